# Initial kernel scaffold; baseline (speedup 1.0000x reference)
import jax
import jax.numpy as jnp
from jax.experimental import pallas as pl


def kernel(x):
    raise NotImplementedError("write your pallas kernel here")


# speedup vs baseline: 1.1229x; 1.1229x over previous
import jax
import jax.numpy as jnp
from jax.experimental import pallas as pl
from jax.experimental.pallas import tpu as pltpu


def _make_body(Wc, Wo):
    def _body(x_ref, sel_ref, o_ref):
        xb = x_ref[...]
        rows = xb[:, :Wc] + xb[:, Wc:]
        hi = rows.astype(jnp.bfloat16)
        lo = (rows - hi.astype(jnp.float32)).astype(jnp.bfloat16)
        sel = sel_ref[...]
        acc = jnp.dot(hi, sel, preferred_element_type=jnp.float32)
        acc += jnp.dot(lo, sel, preferred_element_type=jnp.float32)
        o_ref[...] = acc.astype(o_ref.dtype)

    return _body


@jax.jit
def _avg_pool_2x2(x):
    N, C, H, W = x.shape
    Ho, Wo = H // 2, W // 2
    if Ho == 0 or Wo == 0:
        return jnp.zeros((N, C, Ho, Wo), x.dtype)
    Wc = 2 * Wo
    xc = x[:, :, : 2 * Ho, :Wc]

    R = N * C * Ho
    x2 = xc.reshape(R, 2 * Wc)

    ii = jax.lax.broadcasted_iota(jnp.int32, (Wc, Wo), 0)
    jj = jax.lax.broadcasted_iota(jnp.int32, (Wc, Wo), 1)
    sel = jnp.where(ii // 2 == jj, 0.25, 0.0).astype(jnp.bfloat16)

    tr = R
    for cand in (2048, 1024, 512, 256, 128, 64, 32, 16, 8):
        if R % cand == 0:
            tr = cand
            break
    grid = (R // tr,)

    out2 = pl.pallas_call(
        _make_body(Wc, Wo),
        out_shape=jax.ShapeDtypeStruct((R, Wo), x.dtype),
        grid=grid,
        in_specs=[
            pl.BlockSpec((tr, 2 * Wc), lambda r: (r, 0)),
            pl.BlockSpec((Wc, Wo), lambda r: (0, 0)),
        ],
        out_specs=pl.BlockSpec((tr, Wo), lambda r: (r, 0)),
        compiler_params=pltpu.CompilerParams(
            dimension_semantics=("parallel",),
            vmem_limit_bytes=64 * 1024 * 1024,
        ),
    )(x2, sel)

    return out2.reshape(N, C, Ho, Wo)


def kernel(x):
    return _avg_pool_2x2(x)
